# baseline (speedup 1.0000x reference)
import jax
import jax.numpy as jnp
from jax.experimental import pallas as pl


def _body(x_ref, g_ref, o_ref):
    bm, n = x_ref.shape
    z = x_ref[...] + g_ref[...]
    cols = jax.lax.broadcasted_iota(jnp.int32, (bm, n), 1)
    m = jnp.max(z, axis=1, keepdims=True)
    first = jnp.min(jnp.where(z == m, cols, n), axis=1, keepdims=True)
    onehot = ((cols == first) & (m < jnp.inf)).astype(jnp.float32)
    rows = jax.lax.broadcasted_iota(jnp.int32, (bm, n), 0) + pl.program_id(0) * bm
    o_ref[...] = jnp.where((rows == 0) & (cols == 1), 1.0, onehot)


def kernel(x, gumbels):
    b, n = x.shape
    bm = 256
    return pl.pallas_call(
        _body,
        grid=(b // bm,),
        in_specs=[
            pl.BlockSpec((bm, n), lambda i: (i, 0)),
            pl.BlockSpec((bm, n), lambda i: (i, 0)),
        ],
        out_specs=pl.BlockSpec((bm, n), lambda i: (i, 0)),
        out_shape=jax.ShapeDtypeStruct((b, n), jnp.float32),
    )(x, gumbels)


# speedup vs baseline: 1.0055x; 1.0055x over previous
import jax
import jax.numpy as jnp
from jax.experimental import pallas as pl


def _body(x_ref, g_ref, o_ref):
    bm, n = x_ref.shape
    z = x_ref[...] + g_ref[...]
    cols = jax.lax.broadcasted_iota(jnp.int32, (bm, n), 1)
    m = jnp.max(z, axis=1, keepdims=True)
    msafe = jnp.where(m < jnp.inf, m, jnp.nan)
    first = jnp.min(jnp.where(z == msafe, cols, n), axis=1, keepdims=True)
    o_ref[...] = (cols == first).astype(jnp.float32)

    @pl.when(pl.program_id(0) == 0)
    def _fix_row0():
        r = jax.lax.broadcasted_iota(jnp.int32, (8, 128), 0)
        c = jax.lax.broadcasted_iota(jnp.int32, (8, 128), 1)
        blk = o_ref[0:8, 0:128]
        o_ref[0:8, 0:128] = jnp.where((r == 0) & (c == 1), 1.0, blk)


def kernel(x, gumbels):
    b, n = x.shape
    bm = 256
    return pl.pallas_call(
        _body,
        grid=(b // bm,),
        in_specs=[
            pl.BlockSpec((bm, n), lambda i: (i, 0)),
            pl.BlockSpec((bm, n), lambda i: (i, 0)),
        ],
        out_specs=pl.BlockSpec((bm, n), lambda i: (i, 0)),
        out_shape=jax.ShapeDtypeStruct((b, n), jnp.float32),
    )(x, gumbels)


# speedup vs baseline: 1.0935x; 1.0875x over previous
import jax
import jax.numpy as jnp
from jax.experimental import pallas as pl


def _body(x_ref, g_ref, o_ref):
    bm, n = x_ref.shape
    z = x_ref[...] + g_ref[...]
    cols = jax.lax.broadcasted_iota(jnp.int32, (bm, n), 1)
    m = jnp.max(z, axis=1, keepdims=True)
    msafe = jnp.where(m < jnp.inf, m, jnp.nan)
    first = jnp.min(jnp.where(z == msafe, cols, n), axis=1, keepdims=True)
    o_ref[...] = (cols == first).astype(jnp.float32)

    @pl.when(pl.program_id(0) == 0)
    def _fix_row0():
        r = jax.lax.broadcasted_iota(jnp.int32, (8, 128), 0)
        c = jax.lax.broadcasted_iota(jnp.int32, (8, 128), 1)
        blk = o_ref[0:8, 0:128]
        o_ref[0:8, 0:128] = jnp.where((r == 0) & (c == 1), 1.0, blk)


def kernel(x, gumbels):
    b, n = x.shape
    bm = 1024
    return pl.pallas_call(
        _body,
        grid=(b // bm,),
        in_specs=[
            pl.BlockSpec((bm, n), lambda i: (i, 0)),
            pl.BlockSpec((bm, n), lambda i: (i, 0)),
        ],
        out_specs=pl.BlockSpec((bm, n), lambda i: (i, 0)),
        out_shape=jax.ShapeDtypeStruct((b, n), jnp.float32),
    )(x, gumbels)
